# Initial kernel scaffold; baseline (speedup 1.0000x reference)
import jax
import jax.numpy as jnp
from jax.experimental import pallas as pl


def kernel(x, edge_index, batch, W1, b1, W2, b2, Wc, bc):
    raise NotImplementedError("write your pallas kernel here")


# speedup vs baseline: 12.2612x; 12.2612x over previous
import functools

import jax
import jax.numpy as jnp
from jax import lax
from jax.experimental import pallas as pl
from jax.experimental.pallas import tpu as pltpu
from jax.experimental.pallas import tpu_sc as plsc

N = 10000
E = 320000
D = 128
G = 64
OUT = 10
NP = 10240
NW = 32
KCH = 79
CH = 128
EP = NW * KCH * CH
RPT = NP // 16
DW = 16

_f32 = jnp.float32


def _mm_body(x_ref, w_ref, o_ref):
    o_ref[...] = jnp.dot(x_ref[...], w_ref[...], preferred_element_type=_f32)


def _scale_body(xw_ref, dp_ref, o_ref):
    cnt = dp_ref[0, :, :1] + dp_ref[1, :, :1]
    dinv = lax.rsqrt(cnt + 1.0)
    o_ref[...] = xw_ref[...] * dinv


def _mid_body(p_ref, xw_ref, dp_ref, w2_ref, b1_ref, xw2_ref, y2_ref):
    cnt = dp_ref[0, :, :1] + dp_ref[1, :, :1]
    dinv = lax.rsqrt(cnt + 1.0)
    agg = p_ref[0] + p_ref[1]
    h = jnp.maximum(dinv * agg + (dinv * dinv) * xw_ref[...] + b1_ref[...], 0.0)
    xw2 = jnp.dot(h, w2_ref[...], preferred_element_type=_f32)
    xw2_ref[...] = xw2
    y2_ref[...] = xw2 * dinv


def _final_body(q_ref, xw2_ref, dp_ref, b2_ref, batch_ref, wc_ref, bc_ref, o_ref):
    cnt = dp_ref[0, :, :1] + dp_ref[1, :, :1]
    dinv = lax.rsqrt(cnt + 1.0)
    agg = q_ref[0] + q_ref[1]
    h = jnp.maximum(dinv * agg + (dinv * dinv) * xw2_ref[...] + b2_ref[...], 0.0)
    gids = lax.broadcasted_iota(jnp.int32, (1, G), 1).astype(_f32)
    onehot = (batch_ref[...] == gids).astype(_f32)
    sums = lax.dot_general(onehot, h, (((0,), (0,)), ((), ())),
                           preferred_element_type=_f32)
    counts = jnp.sum(onehot, axis=0)[:, None]
    pooled = sums / jnp.maximum(counts, 1.0)
    o_ref[...] = jnp.dot(pooled, wc_ref[...], preferred_element_type=_f32) + bc_ref[...]


_MESH = plsc.VectorSubcoreMesh(core_axis_name="c", subcore_axis_name="s")


def _deg_body(dst_hbm, dp_hbm, idx_v, ones_v, deg_sh):
    cid = lax.axis_index("c")
    sid = lax.axis_index("s")
    wid = sid * 2 + cid

    def fill0(i, carry):
        ones_v[i, :] = jnp.zeros((16,), _f32)
        return carry
    lax.fori_loop(0, CH, fill0, 0)

    def zrow(k, carry):
        pltpu.sync_copy(ones_v, deg_sh.at[pl.ds(sid * RPT + k * CH, CH)])
        return carry
    lax.fori_loop(0, RPT // CH, zrow, 0)

    def fill1(i, carry):
        ones_v[i, :] = jnp.full((16,), 1.0, _f32)
        return carry
    lax.fori_loop(0, CH, fill1, 0)
    plsc.subcore_barrier()

    pltpu.sync_copy(dst_hbm.at[wid], idx_v)

    def chunk(j, carry):
        pltpu.sync_copy(ones_v, deg_sh.at[idx_v.at[j]], add=True)
        return carry
    lax.fori_loop(0, KCH, chunk, 0)
    plsc.subcore_barrier()

    def wb(k, carry):
        sl = pl.ds(sid * RPT + k * CH, CH)
        pltpu.sync_copy(deg_sh.at[sl], dp_hbm.at[cid, sl])
        return carry
    lax.fori_loop(0, RPT // CH, wb, 0)


def _agg_body(y_hbm, src_hbm, dst_hbm, out_hbm, si_v, di_v, rows_v, agg_sh):
    cid = lax.axis_index("c")
    sid = lax.axis_index("s")
    wid = sid * 2 + cid

    def zfill(i, carry):
        r = i // 8
        col = (i % 8) * 16
        rows_v[r, pl.ds(col, 16)] = jnp.zeros((16,), _f32)
        return carry
    lax.fori_loop(0, CH * 8, zfill, 0)

    def zrow(k, carry):
        pltpu.sync_copy(rows_v, agg_sh.at[pl.ds(sid * RPT + k * CH, CH)])
        return carry
    lax.fori_loop(0, RPT // CH, zrow, 0)
    plsc.subcore_barrier()

    pltpu.sync_copy(src_hbm.at[wid], si_v)
    pltpu.sync_copy(dst_hbm.at[wid], di_v)

    def chunk(j, carry):
        pltpu.sync_copy(y_hbm.at[si_v.at[j]], rows_v)
        pltpu.sync_copy(rows_v, agg_sh.at[di_v.at[j]], add=True)
        return carry
    lax.fori_loop(0, KCH, chunk, 0)
    plsc.subcore_barrier()

    def wb(k, carry):
        sl = pl.ds(sid * RPT + k * CH, CH)
        pltpu.sync_copy(agg_sh.at[sl], out_hbm.at[cid, sl])
        return carry
    lax.fori_loop(0, RPT // CH, wb, 0)


_deg_kernel = functools.partial(
    pl.kernel,
    out_type=jax.ShapeDtypeStruct((2, NP, DW), _f32),
    scratch_types=[
        pltpu.VMEM((KCH, CH), jnp.int32),
        pltpu.VMEM((CH, DW), _f32),
        pltpu.VMEM_SHARED((NP, DW), _f32),
    ],
    mesh=_MESH,
)(_deg_body)

_agg_kernel = functools.partial(
    pl.kernel,
    out_type=jax.ShapeDtypeStruct((2, NP, D), _f32),
    scratch_types=[
        pltpu.VMEM((KCH, CH), jnp.int32),
        pltpu.VMEM((KCH, CH), jnp.int32),
        pltpu.VMEM((CH, D), _f32),
        pltpu.VMEM_SHARED((NP, D), _f32),
    ],
    mesh=_MESH,
)(_agg_body)


def kernel(x, edge_index, batch, W1, b1, W2, b2, Wc, bc):
    src = edge_index[0].astype(jnp.int32)
    dst = edge_index[1].astype(jnp.int32)
    pad = jnp.full((EP - E,), N, jnp.int32)
    src_r = jnp.concatenate([src, pad]).reshape(NW, KCH, CH)
    dst_r = jnp.concatenate([dst, pad]).reshape(NW, KCH, CH)
    x_pad = jnp.concatenate([x, jnp.zeros((NP - N, D), _f32)], axis=0)
    batchf = jnp.concatenate(
        [batch.astype(_f32), jnp.full((NP - N,), float(G), _f32)])[:, None]
    b1r = b1.reshape(1, D)
    b2r = b2.reshape(1, D)
    bcr = bc.reshape(1, OUT)

    xw1 = pl.pallas_call(
        _mm_body,
        out_shape=jax.ShapeDtypeStruct((NP, D), _f32),
    )(x_pad, W1)

    dp = _deg_kernel(dst_r)

    y1 = pl.pallas_call(
        _scale_body,
        out_shape=jax.ShapeDtypeStruct((NP, D), _f32),
    )(xw1, dp)

    p = _agg_kernel(y1, src_r, dst_r)

    xw2, y2 = pl.pallas_call(
        _mid_body,
        out_shape=(jax.ShapeDtypeStruct((NP, D), _f32),
                   jax.ShapeDtypeStruct((NP, D), _f32)),
    )(p, xw1, dp, W2, b1r)

    q = _agg_kernel(y2, src_r, dst_r)

    out = pl.pallas_call(
        _final_body,
        out_shape=jax.ShapeDtypeStruct((G, OUT), _f32),
    )(q, xw2, dp, b2r, batchf, Wc, bcr)
    return out
